# baseline (speedup 1.0000x reference)
import functools

import jax
import jax.numpy as jnp
import numpy as np
from jax.experimental import pallas as pl
from jax.experimental.pallas import tpu as pltpu

_N, _F, _T = 131072, 64, 8
_BN = 4096


def _body(x_ref, thr_ref, s_ref, o_ref):
    x = x_ref[...]
    h = 256
    for jh in range(2):
        xr = jax.lax.dot_general(
            x, s_ref[:, jh * h:(jh + 1) * h],
            dimension_numbers=(((1,), (0,)), ((), ())),
            precision=jax.lax.Precision.HIGHEST,
            preferred_element_type=jnp.float32,
        )
        o_ref[:, jh * h:(jh + 1) * h] = (
            xr > thr_ref[:, jh * h:(jh + 1) * h]).astype(jnp.float32)


@functools.partial(jax.jit, static_argnames=())
def kernel(x, thresholds):
    n, f = x.shape
    t = thresholds.shape[-1]
    thr_flat = thresholds.reshape(1, f * t)
    sel = jnp.asarray(np.repeat(np.eye(f, dtype=np.float32), t, axis=1))
    grid = (n // _BN,)
    out = pl.pallas_call(
        _body,
        grid=grid,
        in_specs=[
            pl.BlockSpec((_BN, f), lambda i: (i, 0)),
            pl.BlockSpec((1, f * t), lambda i: (0, 0)),
            pl.BlockSpec((f, f * t), lambda i: (0, 0)),
        ],
        out_specs=pl.BlockSpec((_BN, f * t), lambda i: (i, 0)),
        out_shape=jax.ShapeDtypeStruct((n, f * t), jnp.float32),
        compiler_params=pltpu.CompilerParams(
            dimension_semantics=("parallel",),
        ),
    )(x, thr_flat, sel)
    return out
